# baseline (speedup 1.0000x reference)
import functools

import jax
import jax.numpy as jnp
from jax import lax
from jax.experimental import pallas as pl
from jax.experimental.pallas import tpu as pltpu
from jax.experimental.pallas import tpu_sc as plsc

N = 10000
D = 128
E = 320000
EB = 2560
E_PAD = EB * 128
NC = 2
NS = 16
NW = NC * NS
KPW = EB // NW
H_CH = 40
K0 = 120
K1 = EB // NS - K0
N_ACC = 10240

_mesh = plsc.VectorSubcoreMesh(core_axis_name="c", subcore_axis_name="s")


@functools.partial(
    pl.kernel,
    mesh=_mesh,
    out_type=jax.ShapeDtypeStruct((NC, N, D), jnp.float32),
    scratch_types=[
        pltpu.VMEM((H_CH, 128), jnp.int32),
        pltpu.VMEM((H_CH, 128), jnp.int32),
        pltpu.VMEM((128, D), jnp.float32),
        pltpu.VMEM((128, D), jnp.float32),
        pltpu.VMEM_SHARED((N_ACC, D), jnp.float32),
        pltpu.SemaphoreType.DMA,
        pltpu.SemaphoreType.DMA,
    ],
)
def _sc_agg(h_hbm, src_hbm, dst_hbm, zeros_hbm, out_hbm,
            src_v, dst_v, rows_a, rows_b, acc, sem_a, sem_b):
    cid = lax.axis_index("c")
    sid = lax.axis_index("s")
    z = N_ACC // NS
    pltpu.sync_copy(zeros_hbm.at[pl.ds(sid * z, z)], acc.at[pl.ds(sid * z, z)])
    plsc.subcore_barrier()

    def run_chunks(base_batches, nchunks):
        for chunk in range(nchunks):
            base = base_batches + chunk * H_CH
            pltpu.sync_copy(src_hbm.at[pl.ds(base, H_CH)], src_v)
            pltpu.sync_copy(dst_hbm.at[pl.ds(base, H_CH)], dst_v)

            pltpu.async_copy(h_hbm.at[src_v.at[0]], rows_a, sem_a)

            def body(jj, carry):
                j0 = jj * 2
                j1 = j0 + 1
                pltpu.async_copy(h_hbm.at[src_v.at[j1]], rows_b, sem_b)
                pltpu.make_async_copy(h_hbm.at[src_v.at[j0]], rows_a,
                                      sem_a).wait()
                pltpu.sync_copy(rows_a, acc.at[dst_v.at[j0]], add=True)
                pltpu.async_copy(h_hbm.at[src_v.at[j0 + 2]], rows_a, sem_a)
                pltpu.make_async_copy(h_hbm.at[src_v.at[j1]], rows_b,
                                      sem_b).wait()
                pltpu.sync_copy(rows_b, acc.at[dst_v.at[j1]], add=True)
                return carry

            lax.fori_loop(0, H_CH // 2 - 1, body, 0)
            pltpu.async_copy(h_hbm.at[src_v.at[H_CH - 1]], rows_b, sem_b)
            pltpu.make_async_copy(h_hbm.at[src_v.at[H_CH - 2]], rows_a,
                                  sem_a).wait()
            pltpu.sync_copy(rows_a, acc.at[dst_v.at[H_CH - 2]], add=True)
            pltpu.make_async_copy(h_hbm.at[src_v.at[H_CH - 1]], rows_b,
                                  sem_b).wait()
            pltpu.sync_copy(rows_b, acc.at[dst_v.at[H_CH - 1]], add=True)

    @pl.when(cid == 0)
    def _c0():
        run_chunks(sid * K0, K0 // H_CH)

    @pl.when(cid == 1)
    def _c1():
        run_chunks(NS * K0 + sid * K1, K1 // H_CH)

    plsc.subcore_barrier()
    pltpu.sync_copy(acc.at[pl.ds(sid * 624, 624)],
                    out_hbm.at[cid, pl.ds(sid * 624, 624)])

    @pl.when(sid == NS - 1)
    def _tail():
        pltpu.sync_copy(acc.at[pl.ds(NS * 624, N - NS * 624)],
                        out_hbm.at[cid, pl.ds(NS * 624, N - NS * 624)])


@functools.partial(
    pl.kernel,
    mesh=_mesh,
    out_type=jax.ShapeDtypeStruct((NC, N, D), jnp.float32),
    scratch_types=[
        pltpu.VMEM((KPW, 128), jnp.int32),
        pltpu.VMEM((128, D), jnp.float32),
        pltpu.VMEM_SHARED((N_ACC, D), jnp.float32),
        pltpu.SemaphoreType.DMA,
    ],
)
def _sc_deg(dst_hbm, zeros_hbm, ones_hbm, out_hbm, dst_v, ones_v, acc, sem):
    cid = lax.axis_index("c")
    sid = lax.axis_index("s")
    wid = sid * NC + cid
    z = N_ACC // NS
    pltpu.sync_copy(zeros_hbm.at[pl.ds(sid * z, z)], acc.at[pl.ds(sid * z, z)])
    pltpu.sync_copy(ones_hbm.at[pl.ds(0, 128)], ones_v)
    pltpu.sync_copy(dst_hbm.at[pl.ds(wid * KPW, KPW)], dst_v)
    plsc.subcore_barrier()

    def body(j, carry):
        pltpu.sync_copy(ones_v, acc.at[dst_v.at[j]], add=True)
        return carry

    lax.fori_loop(0, KPW, body, 0)
    plsc.subcore_barrier()
    pltpu.sync_copy(acc.at[pl.ds(sid * 624, 624)],
                    out_hbm.at[cid, pl.ds(sid * 624, 624)])

    @pl.when(sid == NS - 1)
    def _tail():
        pltpu.sync_copy(acc.at[pl.ds(NS * 624, N - NS * 624)],
                        out_hbm.at[cid, pl.ds(NS * 624, N - NS * 624)])


def _tc_layer_body(act, h_ref, a0_ref, a1_ref, rd_ref, ws_ref, wn_ref, b_ref,
                   o_ref):
    hn = (a0_ref[...] + a1_ref[...]) * rd_ref[...]
    out = (jnp.dot(h_ref[...], ws_ref[...], preferred_element_type=jnp.float32)
           + jnp.dot(hn, wn_ref[...], preferred_element_type=jnp.float32)
           + b_ref[...])
    if act:
        out = jnp.maximum(out, 0.0)
    o_ref[...] = out


def _tc_layer(h, a0, a1, rdeg, ws, wn, b, act):
    bn = 2000
    return pl.pallas_call(
        functools.partial(_tc_layer_body, act),
        grid=(N // bn,),
        in_specs=[
            pl.BlockSpec((bn, D), lambda i: (i, 0)),
            pl.BlockSpec((bn, D), lambda i: (i, 0)),
            pl.BlockSpec((bn, D), lambda i: (i, 0)),
            pl.BlockSpec((bn, 1), lambda i: (i, 0)),
            pl.BlockSpec((D, D), lambda i: (0, 0)),
            pl.BlockSpec((D, D), lambda i: (0, 0)),
            pl.BlockSpec((1, D), lambda i: (0, 0)),
        ],
        out_specs=pl.BlockSpec((bn, D), lambda i: (i, 0)),
        out_shape=jax.ShapeDtypeStruct((N, D), jnp.float32),
    )(h, a0, a1, rdeg, ws, wn, b)


def _tc_layer0_body(h_ref, a0_ref, a1_ref, dp_ref, ws_ref, wn_ref, b_ref,
                    o_ref, rd_ref):
    rd = 1.0 / jnp.maximum(dp_ref[0][:, 0:1] + dp_ref[1][:, 0:1], 1.0)
    rd_ref[...] = rd
    hn = (a0_ref[...] + a1_ref[...]) * rd
    out = (jnp.dot(h_ref[...], ws_ref[...], preferred_element_type=jnp.float32)
           + jnp.dot(hn, wn_ref[...], preferred_element_type=jnp.float32)
           + b_ref[...])
    o_ref[...] = jnp.maximum(out, 0.0)


def _tc_layer0(h, a0, a1, degp, ws, wn, b):
    bn = 2000
    return pl.pallas_call(
        _tc_layer0_body,
        grid=(N // bn,),
        in_specs=[
            pl.BlockSpec((bn, D), lambda i: (i, 0)),
            pl.BlockSpec((bn, D), lambda i: (i, 0)),
            pl.BlockSpec((bn, D), lambda i: (i, 0)),
            pl.BlockSpec((2, bn, D), lambda i: (0, i, 0)),
            pl.BlockSpec((D, D), lambda i: (0, 0)),
            pl.BlockSpec((D, D), lambda i: (0, 0)),
            pl.BlockSpec((1, D), lambda i: (0, 0)),
        ],
        out_specs=[
            pl.BlockSpec((bn, D), lambda i: (i, 0)),
            pl.BlockSpec((bn, 1), lambda i: (i, 0)),
        ],
        out_shape=[
            jax.ShapeDtypeStruct((N, D), jnp.float32),
            jax.ShapeDtypeStruct((N, 1), jnp.float32),
        ],
    )(h, a0, a1, degp, ws, wn, b)


def kernel(features, edge_index, W_self0, W_neigh0, b0, W_self1, W_neigh1, b1,
           W_self2, W_neigh2, b2):
    src = edge_index[0].astype(jnp.int32)
    dst = edge_index[1].astype(jnp.int32)
    pad = E_PAD - E
    src2 = jnp.concatenate([src, jnp.zeros((pad,), jnp.int32)]).reshape(EB, 128)
    dst2 = jnp.concatenate([dst, jnp.full((pad,), N, jnp.int32)]).reshape(EB, 128)
    zeros128 = jnp.zeros((N_ACC, D), jnp.float32)
    ones128 = jnp.ones((128, D), jnp.float32)

    degp = _sc_deg(dst2, zeros128, ones128)

    parts = _sc_agg(features, src2, dst2, zeros128)
    h, rdeg = _tc_layer0(features, parts[0], parts[1], degp,
                         W_self0, W_neigh0, b0.reshape(1, D))
    for ws, wn, b, act in ((W_self1, W_neigh1, b1, True),
                           (W_self2, W_neigh2, b2, False)):
        parts = _sc_agg(h, src2, dst2, zeros128)
        h = _tc_layer(h, parts[0], parts[1], rdeg, ws, wn, b.reshape(1, D), act)
    return h


# speedup vs baseline: 1.1079x; 1.1079x over previous
import functools

import jax
import jax.numpy as jnp
from jax import lax
from jax.experimental import pallas as pl
from jax.experimental.pallas import tpu as pltpu
from jax.experimental.pallas import tpu_sc as plsc

N = 10000
D = 128
E = 320000
EB = 2560
E_PAD = EB * 128
NC = 2
NS = 16
NW = NC * NS
KPW = EB // NW
H_CH = 40
K0 = 120
K1 = EB // NS - K0
N_ACC = 10240

_mesh = plsc.VectorSubcoreMesh(core_axis_name="c", subcore_axis_name="s")


@functools.partial(
    pl.kernel,
    mesh=_mesh,
    out_type=jax.ShapeDtypeStruct((NC, N, D), jnp.float32),
    scratch_types=[
        pltpu.VMEM((H_CH, 128), jnp.int32),
        pltpu.VMEM((H_CH, 128), jnp.int32),
        pltpu.VMEM((128, D), jnp.float32),
        pltpu.VMEM((128, D), jnp.float32),
        pltpu.VMEM_SHARED((N_ACC, D), jnp.float32),
        pltpu.SemaphoreType.DMA,
        pltpu.SemaphoreType.DMA,
    ],
)
def _sc_agg(h_hbm, src_hbm, dst_hbm, zeros_hbm, out_hbm,
            src_v, dst_v, rows_a, rows_b, acc, sem_a, sem_b):
    cid = lax.axis_index("c")
    sid = lax.axis_index("s")
    z = N_ACC // NS
    pltpu.sync_copy(zeros_hbm.at[pl.ds(sid * z, z)], acc.at[pl.ds(sid * z, z)])
    plsc.subcore_barrier()

    def run_chunks(base_batches, nchunks):
        for chunk in range(nchunks):
            base = base_batches + chunk * H_CH
            pltpu.sync_copy(src_hbm.at[pl.ds(base, H_CH)], src_v)
            pltpu.sync_copy(dst_hbm.at[pl.ds(base, H_CH)], dst_v)

            pltpu.async_copy(h_hbm.at[src_v.at[0]], rows_a, sem_a)

            def body(jj, carry):
                j0 = jj * 2
                j1 = j0 + 1
                pltpu.async_copy(h_hbm.at[src_v.at[j1]], rows_b, sem_b)
                pltpu.make_async_copy(h_hbm.at[src_v.at[j0]], rows_a,
                                      sem_a).wait()
                pltpu.sync_copy(rows_a, acc.at[dst_v.at[j0]], add=True)
                pltpu.async_copy(h_hbm.at[src_v.at[j0 + 2]], rows_a, sem_a)
                pltpu.make_async_copy(h_hbm.at[src_v.at[j1]], rows_b,
                                      sem_b).wait()
                pltpu.sync_copy(rows_b, acc.at[dst_v.at[j1]], add=True)
                return carry

            lax.fori_loop(0, H_CH // 2 - 1, body, 0)
            pltpu.async_copy(h_hbm.at[src_v.at[H_CH - 1]], rows_b, sem_b)
            pltpu.make_async_copy(h_hbm.at[src_v.at[H_CH - 2]], rows_a,
                                  sem_a).wait()
            pltpu.sync_copy(rows_a, acc.at[dst_v.at[H_CH - 2]], add=True)
            pltpu.make_async_copy(h_hbm.at[src_v.at[H_CH - 1]], rows_b,
                                  sem_b).wait()
            pltpu.sync_copy(rows_b, acc.at[dst_v.at[H_CH - 1]], add=True)

    @pl.when(cid == 0)
    def _c0():
        run_chunks(sid * K0, K0 // H_CH)

    @pl.when(cid == 1)
    def _c1():
        run_chunks(NS * K0 + sid * K1, K1 // H_CH)

    plsc.subcore_barrier()
    pltpu.sync_copy(acc.at[pl.ds(sid * 624, 624)],
                    out_hbm.at[cid, pl.ds(sid * 624, 624)])

    @pl.when(sid == NS - 1)
    def _tail():
        pltpu.sync_copy(acc.at[pl.ds(NS * 624, N - NS * 624)],
                        out_hbm.at[cid, pl.ds(NS * 624, N - NS * 624)])


@functools.partial(
    pl.kernel,
    mesh=_mesh,
    out_type=jax.ShapeDtypeStruct((NC, N, D), jnp.float32),
    scratch_types=[
        pltpu.VMEM((KPW, 128), jnp.int32),
        pltpu.VMEM((128, D), jnp.float32),
        pltpu.VMEM_SHARED((N_ACC, D), jnp.float32),
        pltpu.SemaphoreType.DMA,
    ],
)
def _sc_deg(dst_hbm, zeros_hbm, ones_hbm, out_hbm, dst_v, ones_v, acc, sem):
    cid = lax.axis_index("c")
    sid = lax.axis_index("s")
    wid = sid * NC + cid
    z = N_ACC // NS
    pltpu.sync_copy(zeros_hbm.at[pl.ds(sid * z, z)], acc.at[pl.ds(sid * z, z)])
    pltpu.sync_copy(ones_hbm.at[pl.ds(0, 128)], ones_v)
    pltpu.sync_copy(dst_hbm.at[pl.ds(wid * KPW, KPW)], dst_v)
    plsc.subcore_barrier()

    def body(j, carry):
        pltpu.sync_copy(ones_v, acc.at[dst_v.at[j]], add=True)
        return carry

    lax.fori_loop(0, KPW, body, 0)
    plsc.subcore_barrier()
    pltpu.sync_copy(acc.at[pl.ds(sid * 624, 624)],
                    out_hbm.at[cid, pl.ds(sid * 624, 624)])

    @pl.when(sid == NS - 1)
    def _tail():
        pltpu.sync_copy(acc.at[pl.ds(NS * 624, N - NS * 624)],
                        out_hbm.at[cid, pl.ds(NS * 624, N - NS * 624)])


def _tc_layer_body(act, h_ref, a0_ref, a1_ref, rd_ref, ws_ref, wn_ref, b_ref,
                   o_ref):
    hn = (a0_ref[...] + a1_ref[...]) * rd_ref[...]
    out = (jnp.dot(h_ref[...], ws_ref[...], preferred_element_type=jnp.float32)
           + jnp.dot(hn, wn_ref[...], preferred_element_type=jnp.float32)
           + b_ref[...])
    if act:
        out = jnp.maximum(out, 0.0)
    o_ref[...] = out


def _tc_layer(h, a0, a1, rdeg, ws, wn, b, act):
    bn = 2000
    return pl.pallas_call(
        functools.partial(_tc_layer_body, act),
        grid=(N // bn,),
        in_specs=[
            pl.BlockSpec((bn, D), lambda i: (i, 0)),
            pl.BlockSpec((bn, D), lambda i: (i, 0)),
            pl.BlockSpec((bn, D), lambda i: (i, 0)),
            pl.BlockSpec((bn, 1), lambda i: (i, 0)),
            pl.BlockSpec((D, D), lambda i: (0, 0)),
            pl.BlockSpec((D, D), lambda i: (0, 0)),
            pl.BlockSpec((1, D), lambda i: (0, 0)),
        ],
        out_specs=pl.BlockSpec((bn, D), lambda i: (i, 0)),
        out_shape=jax.ShapeDtypeStruct((N, D), jnp.float32),
    )(h, a0, a1, rdeg, ws, wn, b)


def _tc_prep(parts):
    bn = 2000

    def body(p_ref, o_ref):
        s = p_ref[0] + p_ref[1]
        o_ref[...] = 1.0 / jnp.maximum(s[:, 0:1], 1.0)

    return pl.pallas_call(
        body,
        grid=(N // bn,),
        in_specs=[pl.BlockSpec((2, bn, D), lambda i: (0, i, 0))],
        out_specs=pl.BlockSpec((bn, 1), lambda i: (i, 0)),
        out_shape=jax.ShapeDtypeStruct((N, 1), jnp.float32),
    )(parts)


def kernel(features, edge_index, W_self0, W_neigh0, b0, W_self1, W_neigh1, b1,
           W_self2, W_neigh2, b2):
    src = edge_index[0].astype(jnp.int32)
    dst = edge_index[1].astype(jnp.int32)
    pad = E_PAD - E
    src2 = jnp.concatenate([src, jnp.zeros((pad,), jnp.int32)]).reshape(EB, 128)
    dst2 = jnp.concatenate([dst, jnp.full((pad,), N, jnp.int32)]).reshape(EB, 128)
    zeros128 = jnp.zeros((N_ACC, D), jnp.float32)
    ones128 = jnp.ones((128, D), jnp.float32)

    degp = _sc_deg(dst2, zeros128, ones128)
    rdeg = _tc_prep(degp)

    h = features
    for ws, wn, b, act in ((W_self0, W_neigh0, b0, True),
                           (W_self1, W_neigh1, b1, True),
                           (W_self2, W_neigh2, b2, False)):
        parts = _sc_agg(h, src2, dst2, zeros128)
        h = _tc_layer(h, parts[0], parts[1], rdeg, ws, wn, b.reshape(1, D), act)
    return h


# speedup vs baseline: 1.1125x; 1.0042x over previous
import functools

import jax
import jax.numpy as jnp
from jax import lax
from jax.experimental import pallas as pl
from jax.experimental.pallas import tpu as pltpu
from jax.experimental.pallas import tpu_sc as plsc

N = 10000
D = 128
E = 320000
EB = 2560
E_PAD = EB * 128
NC = 2
NS = 16
NW = NC * NS
KPW = EB // NW
H_CH = 32
K0 = 128
K1 = EB // NS - K0
N_ACC = 10240

_mesh = plsc.VectorSubcoreMesh(core_axis_name="c", subcore_axis_name="s")


@functools.partial(
    pl.kernel,
    mesh=_mesh,
    out_type=jax.ShapeDtypeStruct((NC, N, D), jnp.float32),
    scratch_types=[
        pltpu.VMEM((H_CH, 128), jnp.int32),
        pltpu.VMEM((H_CH, 128), jnp.int32),
        pltpu.VMEM((128, D), jnp.float32),
        pltpu.VMEM((128, D), jnp.float32),
        pltpu.VMEM_SHARED((N_ACC, D), jnp.float32),
        pltpu.SemaphoreType.DMA,
        pltpu.SemaphoreType.DMA,
    ],
)
def _sc_agg(h_hbm, src_hbm, dst_hbm, zeros_hbm, out_hbm,
            src_v, dst_v, rows_a, rows_b, acc, sem_a, sem_b):
    cid = lax.axis_index("c")
    sid = lax.axis_index("s")
    z = N_ACC // NS
    pltpu.sync_copy(zeros_hbm.at[pl.ds(sid * z, z)], acc.at[pl.ds(sid * z, z)])
    plsc.subcore_barrier()

    def run_chunks(base_batches, nchunks):
        for chunk in range(nchunks):
            base = base_batches + chunk * H_CH
            pltpu.sync_copy(src_hbm.at[pl.ds(base, H_CH)], src_v)
            pltpu.sync_copy(dst_hbm.at[pl.ds(base, H_CH)], dst_v)

            pltpu.async_copy(h_hbm.at[src_v.at[0]], rows_a, sem_a)

            def body(jj, carry):
                j0 = jj * 2
                j1 = j0 + 1
                pltpu.async_copy(h_hbm.at[src_v.at[j1]], rows_b, sem_b)
                pltpu.make_async_copy(h_hbm.at[src_v.at[j0]], rows_a,
                                      sem_a).wait()
                pltpu.sync_copy(rows_a, acc.at[dst_v.at[j0]], add=True)
                pltpu.async_copy(h_hbm.at[src_v.at[j0 + 2]], rows_a, sem_a)
                pltpu.make_async_copy(h_hbm.at[src_v.at[j1]], rows_b,
                                      sem_b).wait()
                pltpu.sync_copy(rows_b, acc.at[dst_v.at[j1]], add=True)
                return carry

            lax.fori_loop(0, H_CH // 2 - 1, body, 0)
            pltpu.async_copy(h_hbm.at[src_v.at[H_CH - 1]], rows_b, sem_b)
            pltpu.make_async_copy(h_hbm.at[src_v.at[H_CH - 2]], rows_a,
                                  sem_a).wait()
            pltpu.sync_copy(rows_a, acc.at[dst_v.at[H_CH - 2]], add=True)
            pltpu.make_async_copy(h_hbm.at[src_v.at[H_CH - 1]], rows_b,
                                  sem_b).wait()
            pltpu.sync_copy(rows_b, acc.at[dst_v.at[H_CH - 1]], add=True)

    @pl.when(cid == 0)
    def _c0():
        run_chunks(sid * K0, K0 // H_CH)

    @pl.when(cid == 1)
    def _c1():
        run_chunks(NS * K0 + sid * K1, K1 // H_CH)

    plsc.subcore_barrier()
    pltpu.sync_copy(acc.at[pl.ds(sid * 624, 624)],
                    out_hbm.at[cid, pl.ds(sid * 624, 624)])

    @pl.when(sid == NS - 1)
    def _tail():
        pltpu.sync_copy(acc.at[pl.ds(NS * 624, N - NS * 624)],
                        out_hbm.at[cid, pl.ds(NS * 624, N - NS * 624)])


@functools.partial(
    pl.kernel,
    mesh=_mesh,
    out_type=jax.ShapeDtypeStruct((NC, N, D), jnp.float32),
    scratch_types=[
        pltpu.VMEM((KPW, 128), jnp.int32),
        pltpu.VMEM((128, D), jnp.float32),
        pltpu.VMEM_SHARED((N_ACC, D), jnp.float32),
        pltpu.SemaphoreType.DMA,
    ],
)
def _sc_deg(dst_hbm, zeros_hbm, ones_hbm, out_hbm, dst_v, ones_v, acc, sem):
    cid = lax.axis_index("c")
    sid = lax.axis_index("s")
    wid = sid * NC + cid
    z = N_ACC // NS
    pltpu.sync_copy(zeros_hbm.at[pl.ds(sid * z, z)], acc.at[pl.ds(sid * z, z)])
    pltpu.sync_copy(ones_hbm.at[pl.ds(0, 128)], ones_v)
    pltpu.sync_copy(dst_hbm.at[pl.ds(wid * KPW, KPW)], dst_v)
    plsc.subcore_barrier()

    def body(j, carry):
        pltpu.sync_copy(ones_v, acc.at[dst_v.at[j]], add=True)
        return carry

    lax.fori_loop(0, KPW, body, 0)
    plsc.subcore_barrier()
    pltpu.sync_copy(acc.at[pl.ds(sid * 624, 624)],
                    out_hbm.at[cid, pl.ds(sid * 624, 624)])

    @pl.when(sid == NS - 1)
    def _tail():
        pltpu.sync_copy(acc.at[pl.ds(NS * 624, N - NS * 624)],
                        out_hbm.at[cid, pl.ds(NS * 624, N - NS * 624)])


def _tc_layer_body(act, h_ref, a0_ref, a1_ref, rd_ref, ws_ref, wn_ref, b_ref,
                   o_ref):
    hn = (a0_ref[...] + a1_ref[...]) * rd_ref[...]
    out = (jnp.dot(h_ref[...], ws_ref[...], preferred_element_type=jnp.float32)
           + jnp.dot(hn, wn_ref[...], preferred_element_type=jnp.float32)
           + b_ref[...])
    if act:
        out = jnp.maximum(out, 0.0)
    o_ref[...] = out


def _tc_layer(h, a0, a1, rdeg, ws, wn, b, act):
    bn = 2000
    return pl.pallas_call(
        functools.partial(_tc_layer_body, act),
        grid=(N // bn,),
        in_specs=[
            pl.BlockSpec((bn, D), lambda i: (i, 0)),
            pl.BlockSpec((bn, D), lambda i: (i, 0)),
            pl.BlockSpec((bn, D), lambda i: (i, 0)),
            pl.BlockSpec((bn, 1), lambda i: (i, 0)),
            pl.BlockSpec((D, D), lambda i: (0, 0)),
            pl.BlockSpec((D, D), lambda i: (0, 0)),
            pl.BlockSpec((1, D), lambda i: (0, 0)),
        ],
        out_specs=pl.BlockSpec((bn, D), lambda i: (i, 0)),
        out_shape=jax.ShapeDtypeStruct((N, D), jnp.float32),
    )(h, a0, a1, rdeg, ws, wn, b)


def _tc_prep(parts):
    bn = 2000

    def body(p_ref, o_ref):
        s = p_ref[0] + p_ref[1]
        o_ref[...] = 1.0 / jnp.maximum(s[:, 0:1], 1.0)

    return pl.pallas_call(
        body,
        grid=(N // bn,),
        in_specs=[pl.BlockSpec((2, bn, D), lambda i: (0, i, 0))],
        out_specs=pl.BlockSpec((bn, 1), lambda i: (i, 0)),
        out_shape=jax.ShapeDtypeStruct((N, 1), jnp.float32),
    )(parts)


def kernel(features, edge_index, W_self0, W_neigh0, b0, W_self1, W_neigh1, b1,
           W_self2, W_neigh2, b2):
    src = edge_index[0].astype(jnp.int32)
    dst = edge_index[1].astype(jnp.int32)
    pad = E_PAD - E
    src2 = jnp.concatenate([src, jnp.zeros((pad,), jnp.int32)]).reshape(EB, 128)
    dst2 = jnp.concatenate([dst, jnp.full((pad,), N, jnp.int32)]).reshape(EB, 128)
    zeros128 = jnp.zeros((N_ACC, D), jnp.float32)
    ones128 = jnp.ones((128, D), jnp.float32)

    degp = _sc_deg(dst2, zeros128, ones128)
    rdeg = _tc_prep(degp)

    h = features
    for ws, wn, b, act in ((W_self0, W_neigh0, b0, True),
                           (W_self1, W_neigh1, b1, True),
                           (W_self2, W_neigh2, b2, False)):
        parts = _sc_agg(h, src2, dst2, zeros128)
        h = _tc_layer(h, parts[0], parts[1], rdeg, ws, wn, b.reshape(1, D), act)
    return h
